# baseline (speedup 1.0000x reference)
import functools

import jax
import jax.numpy as jnp
from jax import lax
from jax.experimental import pallas as pl
from jax.experimental.pallas import tpu as pltpu
from jax.experimental.pallas import tpu_sc as plsc

_NUM_FIELDS = 26
_VOCAB = 100000
_BATCH = 16384
_LANES = 16
_HALF = _BATCH // 2
_NC, _NS = 2, 16

_mesh = plsc.VectorSubcoreMesh(
    core_axis_name="c", subcore_axis_name="s", num_cores=_NC, num_subcores=_NS
)


_FIELDS_PAD = 32


@functools.partial(
    pl.kernel,
    out_type=jax.ShapeDtypeStruct((_NUM_FIELDS, _BATCH), jnp.float32),
    mesh=_mesh,
    scratch_types=[
        pltpu.VMEM((_VOCAB,), jnp.float32),
        pltpu.VMEM((_BATCH,), jnp.int32),
        pltpu.VMEM((_HALF,), jnp.float32),
        pltpu.SemaphoreType.DMA,
        pltpu.SemaphoreType.DMA,
    ],
    compiler_params=pltpu.CompilerParams(needs_layout_passes=False),
)
def _gather_fields(tables_hbm, x_hbm, partial_hbm, table_v, idx_v, out_v, sem_t, sem_x):
    wid = lax.axis_index("s") * _NC + lax.axis_index("c")

    @pl.when(wid < _NUM_FIELDS)
    def _():
        cp_t = pltpu.async_copy(tables_hbm.at[wid], table_v, sem_t)
        cp_x = pltpu.async_copy(x_hbm.at[wid], idx_v, sem_x)
        cp_t.wait()
        cp_x.wait()

        def half(h):
            base = h * _HALF

            @plsc.parallel_loop(0, _HALF // _LANES, 1, unroll=8)
            def _loop(i):
                idx = idx_v[pl.ds(base + i * _LANES, _LANES)]
                out_v[pl.ds(i * _LANES, _LANES)] = plsc.load_gather(table_v, [idx])

            pltpu.sync_copy(out_v, partial_hbm.at[wid, pl.ds(base, _HALF)])

        half(0)
        half(1)


def _combine_body(p_ref, w_ref, b_ref, o_ref):
    p = p_ref[...]
    w = w_ref[...]
    o_ref[...] = jax.nn.sigmoid(jnp.sum(p * w, axis=0, keepdims=True) + b_ref[...])


_combine = pl.pallas_call(
    _combine_body,
    out_shape=jax.ShapeDtypeStruct((1, _BATCH), jnp.float32),
)


def kernel(x, tables, weight, bias):
    tables2d = tables.reshape(_NUM_FIELDS, _VOCAB)
    tables_pad = jnp.pad(tables2d, ((0, _FIELDS_PAD - _NUM_FIELDS), (0, 0)))
    partial = _gather_fields(tables_pad, x.astype(jnp.int32))
    w = weight.reshape(_NUM_FIELDS, 1)
    b = bias.reshape(1, 1)
    return _combine(partial, w, b)[0]
